# baseline (speedup 1.0000x reference)
import jax
import jax.numpy as jnp
from jax import lax
from jax.experimental import pallas as pl
from jax.experimental.pallas import tpu as pltpu

BSZ, NQ, WIDTH, KGT, NEDGE = 4, 1000, 320, 300, 3000
MIN_DIST = 12.0
KP = 384
NQP = 1024

_C0 = (((0,), (0,)), ((), ()))
_STD = (((1,), (0,)), ((), ()))


def _matcher_body(px_ref, py_ref, gx_ref, gy_ref, e0_ref, e1_ref,
                  adj_ref, msk_ref, dist_ref, mp_ref):
    f32, bf16 = jnp.float32, jnp.bfloat16
    step = pl.program_id(0)
    b = jnp.maximum(step - 4, 0)
    lane = lax.broadcasted_iota(jnp.int32, (1, NQP), 1)
    vid = lax.broadcasted_iota(jnp.int32, (1, KP), 1)

    @pl.when(step < 4)
    def _():
        msk_ref[0] = jnp.ones((NQ, NQ), f32)

    @pl.when(step < 4)
    def _():
        adj_ref[0] = jnp.zeros((NQ, NQ), f32)

    @pl.when(step == 4)
    def _():
        dist_ref[...] = (jnp.abs(gx_ref[...] * WIDTH - px_ref[...] * WIDTH)
                         + jnp.abs(gy_ref[...] * WIDTH - py_ref[...] * WIDTH))

        CH = 20
        lane3 = lax.broadcasted_iota(jnp.int32, (1, 1, NQP), 2)

        def chunk(c, carry):
            used, mpacc = carry
            base = c * CH
            rows = dist_ref[pl.ds(base, CH)]
            rowm = jnp.where(used[None] > 0, jnp.inf, rows)
            m = jnp.min(rowm, axis=2, keepdims=True)
            ok = m < MIN_DIST
            cand = jnp.where((rowm == m) & ok, lane3, NQP)
            js = jnp.min(cand, axis=2, keepdims=True)

            okf = jnp.where(ok, 1.0, 0.0)
            fin = []
            for r in range(CH):
                j_r = js[r]
                ok_r = okf[r]
                conflict = jnp.zeros((BSZ, 1), f32)
                for jf in fin:
                    conflict = jnp.maximum(
                        conflict, jnp.where(j_r == jf, 1.0, 0.0))
                need_fix = conflict * ok_r

                def recompute():
                    row = dist_ref[pl.ds(base + r, 1)].reshape(BSZ, NQP)
                    rm = jnp.where(used > 0, jnp.inf, row)
                    m2 = jnp.min(rm, axis=1, keepdims=True)
                    ok2 = m2 < MIN_DIST
                    c2_ = jnp.where((rm == m2) & ok2, lane, NQP)
                    return (jnp.min(c2_, axis=1, keepdims=True),
                            jnp.where(ok2, 1.0, 0.0))

                j_new, ok_new = lax.cond(jnp.max(need_fix) > 0,
                                         recompute, lambda: (j_r, ok_r))
                fix = need_fix > 0
                j_f = jnp.where(fix, j_new, j_r)
                ok_f = jnp.where(fix, ok_new, ok_r)
                j_f = jnp.where(ok_f > 0, j_f, NQP)
                fin.append(j_f)
                used = jnp.where(lane == j_f, 1.0, used)
                mpacc = jnp.where((vid == base + r) & (ok_f > 0),
                                  j_f.astype(f32), mpacc)
            return used, mpacc

        _, mpacc = lax.fori_loop(
            0, KGT // CH, chunk,
            (jnp.zeros((BSZ, NQP), f32), jnp.full((BSZ, KP), -1.0, f32)))
        mp_ref[...] = mpacc

    @pl.when(step >= 4)
    def _dense():
        _dense_stages(b, adj_ref, e0_ref, e1_ref, dist_ref, mp_ref,
                      lane, vid)


def _dense_stages(b, adj_ref, e0_ref, e1_ref, dist_ref, mp_ref, lane, vid):
    f32, bf16 = jnp.float32, jnp.bfloat16
    r2 = lax.broadcasted_iota(jnp.int32, (KP, KP), 0)
    c2 = lax.broadcasted_iota(jnp.int32, (KP, KP), 1)
    eye = (r2 == c2).astype(f32)

    mp_row = mp_ref[pl.ds(b, 1), :]
    matched = (mp_row >= 0).astype(f32)
    qcol = lax.broadcasted_iota(jnp.int32, (NQP, 1), 0).astype(f32)
    PT = (qcol == mp_row).astype(bf16)

    e0 = e0_ref[0]
    e1 = e1_ref[0]
    oh0 = (e0 == vid).astype(bf16)
    oh1 = (e1 == vid).astype(bf16)
    adjv = jnp.minimum(
        lax.dot_general(oh0, oh1, _C0, preferred_element_type=f32), 1.0)

    ones_col = jnp.ones((KP, 1), f32)
    rowsum = jnp.dot(adjv, ones_col, preferred_element_type=f32)
    colsum = lax.dot_general(adjv, ones_col, _C0, preferred_element_type=f32)
    present = ((rowsum + colsum) > 0).astype(f32)

    le = (c2 <= r2).astype(f32)
    v2i = jnp.dot(le, present, preferred_element_type=f32) - 1.0
    Q = ((v2i == vid.astype(f32)) & (present > 0)).astype(bf16)

    adjv_bf = adjv.astype(bf16)
    m1 = lax.dot_general(Q, adjv_bf, _C0, preferred_element_type=f32)
    A = jnp.minimum(
        lax.dot_general(m1.astype(bf16), Q, _STD, preferred_element_type=f32),
        1.0)

    removed = 1.0 - matched
    S = jnp.minimum(A * removed + eye, 1.0).astype(bf16)
    for _ in range(9):
        S = jnp.minimum(
            lax.dot_general(S, S, _STD, preferred_element_type=f32),
            1.0).astype(bf16)
    reach = lax.dot_general(S, A.astype(bf16), _STD, preferred_element_type=f32)
    keep = (reach > 0).astype(bf16)

    t = lax.dot_general(PT, keep, _STD, preferred_element_type=f32)
    outm = lax.dot_general(t.astype(bf16), PT, (((1,), (1,)), ((), ())),
                           preferred_element_type=f32)
    adj_ref[0] = outm[:NQ, :NQ]


def kernel(pred_nodes, pred_heatmaps, gt_nodes, edges):
    del pred_heatmaps
    f32 = jnp.float32
    pad = jnp.full((BSZ, NQP - NQ), 1e9, f32)
    px = jnp.concatenate([pred_nodes[:, :, 0], pad], axis=1).reshape(1, BSZ, NQP)
    py = jnp.concatenate([pred_nodes[:, :, 1], pad], axis=1).reshape(1, BSZ, NQP)
    gx = gt_nodes[:, :, 0].T.reshape(KGT, BSZ, 1)
    gy = gt_nodes[:, :, 1].T.reshape(KGT, BSZ, 1)
    e0 = edges[:, :, 0].reshape(BSZ, NEDGE, 1)
    e1 = edges[:, :, 1].reshape(BSZ, NEDGE, 1)

    full = lambda shape: pl.BlockSpec(shape, lambda s: (0, 0, 0))
    perb = lambda shape: pl.BlockSpec(
        (1,) + shape, lambda s: (jnp.maximum(s - 4, 0), 0, 0))

    adj, msk = pl.pallas_call(
        _matcher_body,
        grid=(2 * BSZ,),
        in_specs=[
            full((1, BSZ, NQP)), full((1, BSZ, NQP)),
            full((KGT, BSZ, 1)), full((KGT, BSZ, 1)),
            perb((NEDGE, 1)), perb((NEDGE, 1)),
        ],
        out_specs=[
            pl.BlockSpec((1, NQ, NQ), lambda s: (jnp.maximum(s - 4, 0), 0, 0)),
            pl.BlockSpec((1, NQ, NQ), lambda s: (jnp.minimum(s, 3), 0, 0)),
        ],
        out_shape=[
            jax.ShapeDtypeStruct((BSZ, NQ, NQ), f32),
            jax.ShapeDtypeStruct((BSZ, NQ, NQ), f32),
        ],
        scratch_shapes=[
            pltpu.VMEM((KGT, BSZ, NQP), f32),
            pltpu.VMEM((BSZ, KP), f32),
        ],
    )(px, py, gx, gy, e0, e1)
    return (adj, msk)


# speedup vs baseline: 1.0450x; 1.0450x over previous
import jax
import jax.numpy as jnp
from jax import lax
from jax.experimental import pallas as pl
from jax.experimental.pallas import tpu as pltpu

BSZ, NQ, WIDTH, KGT, NEDGE = 4, 1000, 320, 300, 3000
MIN_DIST = 12.0
KP = 384
NQP = 1024

_C0 = (((0,), (0,)), ((), ()))
_STD = (((1,), (0,)), ((), ()))


def _matcher_body(px_ref, py_ref, gx_ref, gy_ref, e0_ref, e1_ref,
                  adj_ref, msk_ref, dist_ref, mp_ref):
    f32, bf16 = jnp.float32, jnp.bfloat16
    b = pl.program_id(0)
    lane = lax.broadcasted_iota(jnp.int32, (1, NQP), 1)
    vid = lax.broadcasted_iota(jnp.int32, (1, KP), 1)

    @pl.when(b == 0)
    def _():
        dist_ref[...] = (jnp.abs(gx_ref[...] * WIDTH - px_ref[...] * WIDTH)
                         + jnp.abs(gy_ref[...] * WIDTH - py_ref[...] * WIDTH))

        CH = 20
        lane3 = lax.broadcasted_iota(jnp.int32, (1, 1, NQP), 2)

        def chunk(c, carry):
            used, mpacc = carry
            base = c * CH
            rows = dist_ref[pl.ds(base, CH)]
            rowm = jnp.where(used[None] > 0, jnp.inf, rows)
            m = jnp.min(rowm, axis=2, keepdims=True)
            ok = m < MIN_DIST
            cand = jnp.where((rowm == m) & ok, lane3, NQP)
            js = jnp.min(cand, axis=2, keepdims=True)

            okf = jnp.where(ok, 1.0, 0.0)
            fin = []
            for r in range(CH):
                j_r = js[r]
                ok_r = okf[r]
                conflict = jnp.zeros((BSZ, 1), f32)
                for jf in fin:
                    conflict = jnp.maximum(
                        conflict, jnp.where(j_r == jf, 1.0, 0.0))
                need_fix = conflict * ok_r

                def recompute():
                    row = dist_ref[pl.ds(base + r, 1)].reshape(BSZ, NQP)
                    rm = jnp.where(used > 0, jnp.inf, row)
                    m2 = jnp.min(rm, axis=1, keepdims=True)
                    ok2 = m2 < MIN_DIST
                    c2_ = jnp.where((rm == m2) & ok2, lane, NQP)
                    return (jnp.min(c2_, axis=1, keepdims=True),
                            jnp.where(ok2, 1.0, 0.0))

                j_new, ok_new = lax.cond(jnp.max(need_fix) > 0,
                                         recompute, lambda: (j_r, ok_r))
                fix = need_fix > 0
                j_f = jnp.where(fix, j_new, j_r)
                ok_f = jnp.where(fix, ok_new, ok_r)
                j_f = jnp.where(ok_f > 0, j_f, NQP)
                fin.append(j_f)
                used = jnp.where(lane == j_f, 1.0, used)
                mpacc = jnp.where((vid == base + r) & (ok_f > 0),
                                  j_f.astype(f32), mpacc)
            return used, mpacc

        _, mpacc = lax.fori_loop(
            0, KGT // CH, chunk,
            (jnp.zeros((BSZ, NQP), f32), jnp.full((BSZ, KP), -1.0, f32)))
        mp_ref[...] = mpacc

    r2 = lax.broadcasted_iota(jnp.int32, (KP, KP), 0)
    c2 = lax.broadcasted_iota(jnp.int32, (KP, KP), 1)
    eye = (r2 == c2).astype(f32)

    mp_row = mp_ref[pl.ds(b, 1), :]
    matched = (mp_row >= 0).astype(f32)
    qcol = lax.broadcasted_iota(jnp.int32, (NQP, 1), 0).astype(f32)
    PT = (qcol == mp_row).astype(bf16)

    e0 = e0_ref[0]
    e1 = e1_ref[0]
    oh0 = (e0 == vid).astype(bf16)
    oh1 = (e1 == vid).astype(bf16)
    adjv = jnp.minimum(
        lax.dot_general(oh0, oh1, _C0, preferred_element_type=f32), 1.0)

    ones_col = jnp.ones((KP, 1), f32)
    rowsum = jnp.dot(adjv, ones_col, preferred_element_type=f32)
    colsum = lax.dot_general(adjv, ones_col, _C0, preferred_element_type=f32)
    present = ((rowsum + colsum) > 0).astype(f32)

    le = (c2 <= r2).astype(f32)
    v2i = jnp.dot(le, present, preferred_element_type=f32) - 1.0
    Q = ((v2i == vid.astype(f32)) & (present > 0)).astype(bf16)

    adjv_bf = adjv.astype(bf16)
    m1 = lax.dot_general(Q, adjv_bf, _C0, preferred_element_type=f32)
    A = jnp.minimum(
        lax.dot_general(m1.astype(bf16), Q, _STD, preferred_element_type=f32),
        1.0)

    removed = 1.0 - matched
    S = jnp.minimum(A * removed + eye, 1.0).astype(bf16)
    for _ in range(9):
        S = jnp.minimum(
            lax.dot_general(S, S, _STD, preferred_element_type=f32),
            1.0).astype(bf16)
    reach = lax.dot_general(S, A.astype(bf16), _STD, preferred_element_type=f32)
    keep = (reach > 0).astype(bf16)

    t = lax.dot_general(PT, keep, _STD, preferred_element_type=f32)
    outm = lax.dot_general(t.astype(bf16), PT, (((1,), (1,)), ((), ())),
                           preferred_element_type=f32)
    adj_ref[0] = outm[:NQ, :NQ]
    msk_ref[0] = jnp.ones((NQ, NQ), f32)


def kernel(pred_nodes, pred_heatmaps, gt_nodes, edges):
    del pred_heatmaps
    f32 = jnp.float32
    pad = jnp.full((BSZ, NQP - NQ), 1e9, f32)
    px = jnp.concatenate([pred_nodes[:, :, 0], pad], axis=1).reshape(1, BSZ, NQP)
    py = jnp.concatenate([pred_nodes[:, :, 1], pad], axis=1).reshape(1, BSZ, NQP)
    gx = gt_nodes[:, :, 0].T.reshape(KGT, BSZ, 1)
    gy = gt_nodes[:, :, 1].T.reshape(KGT, BSZ, 1)
    e0 = edges[:, :, 0].reshape(BSZ, NEDGE, 1)
    e1 = edges[:, :, 1].reshape(BSZ, NEDGE, 1)

    full = lambda shape: pl.BlockSpec(shape, lambda b: (0, 0, 0))
    perb = lambda shape: pl.BlockSpec((1,) + shape, lambda b: (b, 0, 0))

    adj, msk = pl.pallas_call(
        _matcher_body,
        grid=(BSZ,),
        in_specs=[
            full((1, BSZ, NQP)), full((1, BSZ, NQP)),
            full((KGT, BSZ, 1)), full((KGT, BSZ, 1)),
            perb((NEDGE, 1)), perb((NEDGE, 1)),
        ],
        out_specs=[perb((NQ, NQ)), perb((NQ, NQ))],
        out_shape=[
            jax.ShapeDtypeStruct((BSZ, NQ, NQ), f32),
            jax.ShapeDtypeStruct((BSZ, NQ, NQ), f32),
        ],
        scratch_shapes=[
            pltpu.VMEM((KGT, BSZ, NQP), f32),
            pltpu.VMEM((BSZ, KP), f32),
        ],
    )(px, py, gx, gy, e0, e1)
    return (adj, msk)
